# baseline (speedup 1.0000x reference)
import jax
import jax.numpy as jnp
from jax.experimental import pallas as pl
from jax.experimental.pallas import tpu as pltpu

_N = 4096
_D = 64
_SIZES = (128, 384, 1024, 1024, 1024, 384, 128)
_MAXC = max(_SIZES)
_OFFS = tuple(sum(_SIZES[:k]) for k in range(len(_SIZES)))


def _node_attention(adj_hbm, emb_ref, hv_ref, out_ref, buf, sem):
    emb = emb_ref[:]
    score = jnp.dot(emb, hv_ref[:],
                    preferred_element_type=jnp.float32)
    w = jnp.exp(score - jnp.max(score))
    rhs = jnp.concatenate([emb * w, w], axis=1)

    def copy_chunk(k, slot):
        return pltpu.make_async_copy(
            adj_hbm.at[pl.ds(_OFFS[k], _SIZES[k]), :],
            buf.at[slot, pl.ds(0, _SIZES[k]), :],
            sem.at[slot],
        )

    copy_chunk(0, 0).start()
    for k in range(len(_SIZES)):
        slot = k % 2
        if k + 1 < len(_SIZES):
            copy_chunk(k + 1, 1 - slot).start()
        copy_chunk(k, slot).wait()
        a = buf[slot, pl.ds(0, _SIZES[k]), :]
        acc = jnp.dot(a, rhs,
                      preferred_element_type=jnp.float32)
        out_ref[pl.ds(_OFFS[k], _SIZES[k]), :] = acc[:, :-1] / acc[:, -1:]


@jax.jit
def kernel(emb, adj, H_v):
    n, d = emb.shape
    return pl.pallas_call(
        _node_attention,
        in_specs=[
            pl.BlockSpec(memory_space=pltpu.MemorySpace.HBM),
            pl.BlockSpec(memory_space=pltpu.MemorySpace.VMEM),
            pl.BlockSpec(memory_space=pltpu.MemorySpace.VMEM),
        ],
        out_specs=pl.BlockSpec(memory_space=pltpu.MemorySpace.VMEM),
        out_shape=jax.ShapeDtypeStruct((n, d), jnp.float32),
        scratch_shapes=[
            pltpu.VMEM((2, _MAXC, _N), jnp.float32),
            pltpu.SemaphoreType.DMA((2,)),
        ],
    )(adj, emb, H_v)
